# baseline (speedup 1.0000x reference)
import functools

import jax
import jax.numpy as jnp
from jax import lax
from jax.experimental import pallas as pl
from jax.experimental.pallas import tpu as pltpu
from jax.experimental.pallas import tpu_sc as plsc

VOCAB = 1000000
D = 64
SEQ = 200
BATCH = 4096
R = BATCH * SEQ
L = 16
NC, NS = 2, 16
NW = NC * NS

N_VT = VOCAB // 128
A_PAIRS = ((N_VT + NW - 1) // NW + 1) // 2

SEQP = 256
G_SPLIT = ((0, 128), (128, 128))
SEQ_PER_W = BATCH // NW
CH_PER_GRP = 4


def _positional_encoding():
    i = jnp.arange(0, D, 2) / D
    pos = jnp.arange(0, SEQ)[:, None].astype(jnp.float32)
    angle_freq = jnp.exp(i * -jnp.log(jnp.array(10000.0)))
    out = jnp.zeros((SEQ, D), dtype=jnp.float32)
    out = out.at[:, 0::2].set(jnp.sin(pos * angle_freq))
    out = out.at[:, 1::2].set(jnp.cos(pos * angle_freq))
    return out


def _transpose_table(table_t, tail):
    mesh = plsc.VectorSubcoreMesh(core_axis_name="c", subcore_axis_name="s")

    @functools.partial(
        pl.kernel,
        out_type=jax.ShapeDtypeStruct((VOCAB, 128), jnp.float32),
        mesh=mesh,
        compiler_params=pltpu.CompilerParams(needs_layout_passes=False),
        scratch_types=[
            pltpu.VMEM((2, D, 128), jnp.float32),
            pltpu.VMEM((2, 128, 128), jnp.float32),
            pltpu.SemaphoreType.DMA,
            pltpu.SemaphoreType.DMA,
            pltpu.SemaphoreType.DMA,
            pltpu.SemaphoreType.DMA,
        ],
    )
    def body(tt_hbm, tail_hbm, scr_hbm, in_v, st_v, g0, g1, o0, o1):
        wid = lax.axis_index("s") * NC + lax.axis_index("c")
        gsems = (g0, g1)
        osems = (o0, o1)
        n_units = (N_VT - wid + NW - 1) // NW
        row_vecs = [lax.iota(jnp.int32, L) + (dg * L) for dg in range(D // L)]

        def unit_col(k):
            return pl.multiple_of((wid + k * NW) * 128, 128)

        def fire_read(k, nb):
            pltpu.async_copy(
                tt_hbm.at[:, pl.ds(unit_col(k), 128)], in_v.at[nb],
                gsems[nb])

        def handle(k, nb):
            @pl.when(k < n_units)
            def _():
                pltpu.make_async_copy(
                    tt_hbm.at[:, pl.ds(unit_col(k), 128)], in_v.at[nb],
                    gsems[nb]).wait()

                @pl.when(k >= 2)
                def _():
                    pltpu.make_async_copy(
                        st_v.at[nb],
                        scr_hbm.at[pl.ds(unit_col(k - 2), 128), :],
                        osems[nb]).wait()

                def trans_body(t, carry):
                    tvec = jnp.full((L,), t, dtype=jnp.int32)
                    for dg in range(D // L):
                        v = plsc.load_gather(
                            in_v.at[nb], [row_vecs[dg], tvec])
                        st_v[nb, t, pl.ds(dg * L, L)] = v
                    return carry

                lax.fori_loop(0, 128, trans_body, 0, unroll=8)

                pltpu.async_copy(
                    st_v.at[nb], scr_hbm.at[pl.ds(unit_col(k), 128), :],
                    osems[nb])

            @pl.when(k + 1 < n_units)
            def _():
                fire_read(k + 1, 1 - nb)

        @pl.when(0 < n_units)
        def _():
            fire_read(0, 0)

        def pair_body(p, carry):
            handle(2 * p, 0)
            handle(2 * p + 1, 1)
            return carry

        lax.fori_loop(0, A_PAIRS, pair_body, 0)

        n_even = (n_units + 1) // 2
        n_odd = n_units // 2

        @pl.when(n_even > 0)
        def _():
            pltpu.make_async_copy(
                st_v.at[0], scr_hbm.at[pl.ds(unit_col(0), 128), :],
                osems[0]).wait()

        @pl.when(n_odd > 0)
        def _():
            pltpu.make_async_copy(
                st_v.at[1], scr_hbm.at[pl.ds(unit_col(0), 128), :],
                osems[1]).wait()

        @pl.when(wid == 0)
        def _():
            pltpu.sync_copy(tail_hbm, scr_hbm.at[pl.ds(N_VT * 128, 64), :])

    return body(table_t, tail)


def _gather_embed(idx, scr, pos):
    mesh = plsc.VectorSubcoreMesh(core_axis_name="c", subcore_axis_name="s")

    @functools.partial(
        pl.kernel,
        out_type=jax.ShapeDtypeStruct((R, 128), jnp.float32),
        mesh=mesh,
        compiler_params=pltpu.CompilerParams(needs_layout_passes=False),
        scratch_types=[
            pltpu.VMEM((CH_PER_GRP * SEQP,), jnp.int32),
            pltpu.VMEM((2, 2 * 128, 128), jnp.float32),
            pltpu.VMEM((SEQ, 128), jnp.float32),
            pltpu.SemaphoreType.DMA,
            pltpu.SemaphoreType.DMA,
            pltpu.SemaphoreType.DMA,
            pltpu.SemaphoreType.DMA,
        ],
    )
    def body(idx_hbm, scr_hbm, pos_hbm, out_hbm,
             idx_v, rows_v, pos_v, g0, g1, o0, o1):
        wid = lax.axis_index("s") * NC + lax.axis_index("c")
        gsems = (g0, g1)
        osems = (o0, o1)
        seq0 = wid * SEQ_PER_W
        pltpu.sync_copy(pos_hbm, pos_v)

        def stage_idx(c):
            base = pl.multiple_of(
                (seq0 + (c // CH_PER_GRP) * CH_PER_GRP) * SEQP, 128)
            pltpu.sync_copy(
                idx_hbm.at[pl.ds(base, CH_PER_GRP * SEQP)], idx_v)

        def fire_gathers(c, nb):
            ls = lax.rem(c, CH_PER_GRP) * SEQP
            for i, (off, sz) in enumerate(G_SPLIT):
                pltpu.async_copy(
                    scr_hbm.at[idx_v.at[pl.ds(ls + off, sz)]],
                    rows_v.at[nb, pl.ds(i * 128, sz)],
                    gsems[nb],
                )

        def row_slice(c):
            return pl.ds(
                pl.multiple_of((seq0 + c) * SEQ, 8), SEQ)

        def handle(c, nb):
            pltpu.make_async_copy(
                out_hbm.at[pl.ds(0, 2 * 128)], rows_v.at[nb],
                gsems[nb]).wait()

            for j in range(D // L):
                jo = j * L

                def add_body(p, carry):
                    rows_v[nb, p, pl.ds(jo, L)] = (
                        rows_v[nb, p, pl.ds(jo, L)] + pos_v[p, pl.ds(jo, L)])
                    return carry

                lax.fori_loop(0, SEQ, add_body, 0, unroll=4)

            @pl.when(c > 0)
            def _():
                pltpu.make_async_copy(
                    rows_v.at[1 - nb, pl.ds(0, SEQ)],
                    out_hbm.at[row_slice(c - 1)], osems[1 - nb]).wait()

            pltpu.async_copy(rows_v.at[nb, pl.ds(0, SEQ)],
                             out_hbm.at[row_slice(c)], osems[nb])

            @pl.when(c + 1 < SEQ_PER_W)
            def _():
                @pl.when(lax.rem(c + 1, CH_PER_GRP) == 0)
                def _():
                    stage_idx(c + 1)
                fire_gathers(c + 1, 1 - nb)

        stage_idx(0)
        fire_gathers(0, 0)

        def pair_body(g, carry):
            handle(2 * g, 0)
            handle(2 * g + 1, 1)
            return carry

        lax.fori_loop(0, SEQ_PER_W // 2, pair_body, 0)

        pltpu.make_async_copy(
            rows_v.at[1, pl.ds(0, SEQ)],
            out_hbm.at[row_slice(SEQ_PER_W - 1)], osems[1]).wait()

    return body(idx, scr, pos)


def kernel(inputs, table):
    idx = jnp.pad(inputs.astype(jnp.int32),
                  ((0, 0), (0, SEQP - SEQ))).reshape(-1)
    table_t = table.T
    tail = jnp.pad(table[VOCAB - 64:], ((0, 0), (0, D)))
    pos = jnp.pad(_positional_encoding(), ((0, 0), (0, 128 - D)))
    scr = _transpose_table(table_t, tail)
    out2d = _gather_embed(idx, scr, pos)
    return out2d[:, :D].reshape(BATCH, SEQ, D)


# speedup vs baseline: 11.6138x; 11.6138x over previous
import functools

import jax
import jax.numpy as jnp
from jax import lax
from jax.experimental import pallas as pl
from jax.experimental.pallas import tpu as pltpu
from jax.experimental.pallas import tpu_sc as plsc

VOCAB = 1000000
D = 64
SEQ = 200
BATCH = 4096
L = 16
NC, NS = 2, 16
NW = NC * NS

K_SEQ = 4
G_SPLIT = ((0, 104), (104, 96))
SEQ_PER_W = BATCH // NW
N_CHUNKS = SEQ_PER_W // K_SEQ
N_GROUPS = N_CHUNKS // 2
D_CH = D // L


def _positional_encoding():
    i = jnp.arange(0, D, 2) / D
    pos = jnp.arange(0, SEQ)[:, None].astype(jnp.float32)
    angle_freq = jnp.exp(i * -jnp.log(jnp.array(10000.0)))
    out = jnp.zeros((SEQ, D), dtype=jnp.float32)
    out = out.at[:, 0::2].set(jnp.sin(pos * angle_freq))
    out = out.at[:, 1::2].set(jnp.cos(pos * angle_freq))
    return out


def _sc_embed(idx, table, pos):
    mesh = plsc.VectorSubcoreMesh(core_axis_name="c", subcore_axis_name="s")

    @functools.partial(
        pl.kernel,
        out_type=jax.ShapeDtypeStruct((BATCH * SEQ, 128), jnp.float32),
        mesh=mesh,
        compiler_params=pltpu.CompilerParams(use_tc_tiling_on_sc=False),
        scratch_types=[
            pltpu.VMEM((2, 8, SEQ), jnp.int32),
            pltpu.VMEM((2, K_SEQ * SEQ, D), jnp.float32),
            pltpu.VMEM((SEQ, D), jnp.float32),
            pltpu.SemaphoreType.DMA,
            pltpu.SemaphoreType.DMA,
            pltpu.SemaphoreType.DMA,
            pltpu.SemaphoreType.DMA,
        ],
    )
    def body(idx_hbm, table_hbm, pos_hbm, out_hbm,
             idx_v, rows_v, pos_v, gsem0, gsem1, osem0, osem1):
        wid = lax.axis_index("s") * NC + lax.axis_index("c")
        seq0 = wid * SEQ_PER_W
        gsems = (gsem0, gsem1)
        osems = (osem0, osem1)
        pltpu.sync_copy(pos_hbm, pos_v)

        def stage_idx(g):
            base = pl.multiple_of(seq0 + g * 8, 8)
            pltpu.sync_copy(idx_hbm.at[pl.ds(base, 8)], idx_v.at[lax.rem(g, 2)])

        def fire_gathers(c, nb):
            pg = lax.rem(c // 2, 2)
            ls = lax.rem(c, 2) * K_SEQ
            for s in range(K_SEQ):
                for off, sz in G_SPLIT:
                    pltpu.async_copy(
                        table_hbm.at[idx_v.at[pg, ls + s, pl.ds(off, sz)]],
                        rows_v.at[nb, pl.ds(s * SEQ + off, sz)],
                        gsems[nb],
                    )

        CROWS = K_SEQ * SEQ

        def chunk_slice(c):
            base = pl.multiple_of((seq0 + c * K_SEQ) * SEQ, CROWS)
            return (pl.ds(base, CROWS), pl.ds(0, D))

        def handle(c, nb):
            pltpu.make_async_copy(
                out_hbm.at[chunk_slice(c)[0], chunk_slice(c)[1]], rows_v.at[nb], gsems[nb]).wait()

            for j in range(D_CH):
                jo = j * L

                def add_body(p, carry):
                    pv = pos_v[p, pl.ds(jo, L)]
                    for s in range(K_SEQ):
                        r = s * SEQ + p
                        rows_v[nb, r, pl.ds(jo, L)] = (
                            rows_v[nb, r, pl.ds(jo, L)] + pv)
                    return carry

                lax.fori_loop(0, SEQ, add_body, 0, unroll=2)

            @pl.when(c > 0)
            def _():
                pltpu.make_async_copy(
                    rows_v.at[1 - nb], out_hbm.at[chunk_slice(c - 1)[0], chunk_slice(c - 1)[1]],
                    osems[1 - nb]).wait()

            pltpu.async_copy(rows_v.at[nb], out_hbm.at[chunk_slice(c)[0], chunk_slice(c)[1]],
                             osems[nb])

            @pl.when(c + 1 < N_CHUNKS)
            def _():
                @pl.when(lax.rem(c + 1, 2) == 0)
                def _():
                    stage_idx((c + 1) // 2)
                fire_gathers(c + 1, 1 - nb)

        stage_idx(0)
        fire_gathers(0, 0)

        def pair_body(g, carry):
            handle(2 * g, 0)
            handle(2 * g + 1, 1)
            return carry

        lax.fori_loop(0, N_GROUPS, pair_body, 0)

        pltpu.make_async_copy(
            rows_v.at[1], out_hbm.at[chunk_slice(N_CHUNKS - 1)[0], chunk_slice(N_CHUNKS - 1)[1]], osem1).wait()

    return body(idx, table, pos)


def kernel(inputs, table):
    idx = inputs.astype(jnp.int32)
    pos = _positional_encoding()
    out2d = _sc_embed(idx, table, pos)
    return out2d[:, :D].reshape(BATCH, SEQ, D)
